# baseline (speedup 1.0000x reference)
import functools

import jax
import jax.numpy as jnp
from jax.experimental import pallas as pl
from jax.experimental.pallas import tpu as pltpu

_SEL_RANKS = frozenset((7, 4, 16, 19, 2, 5, 3, 6, 18, 10))

_K = 20
_NEG = 10
_MASKV = -1000000000.0
_NINF = float("-inf")
_RB = 32
_D = 4
_G = 4
_RBX = 8


def _fine_loss(e_ref, m_ref):
    e = e_ref[...]
    w = 1.0 / jnp.clip(e[2:3, :], 0.0001, None)
    per = w * (e[0:1, :] * e[0:1, :] + e[1:2, :] * e[1:2, :])
    mk = m_ref[...]
    return jnp.sum(per * mk) / jnp.maximum(jnp.sum(mk), 1.0)


def _emit_outputs(acc, e_ref, m_ref, o_tot, o_c, o_f, P):
    loss_c = acc[0] / (P * float(_NEG))
    loss_f = _fine_loss(e_ref, m_ref)
    o_tot[...] = jnp.reshape(1.0 * loss_c + 0.5 * loss_f, (1, 1))
    o_c[...] = jnp.reshape(loss_c, (1, 1))
    o_f[...] = jnp.reshape(loss_f, (1, 1))


def _fast_body(rowid_ref, *refs, S, P):
    sims = refs[:_RB]
    j_ref, e_ref, m_ref = refs[_RB:_RB + 3]
    o_tot, o_c, o_f, o_bad = refs[_RB + 3:_RB + 7]
    acc = refs[_RB + 7]

    p = pl.program_id(0)

    rows = jnp.concatenate(
        [jnp.reshape(s[...], (1, S)) for s in sims], axis=0)
    jv = j_ref[...]
    iota = jax.lax.broadcasted_iota(jnp.int32, (_RB, S), 1)
    isj = iota == jv
    pos = jnp.sum(jnp.where(isj, rows, 0.0), axis=1, keepdims=True)
    x = jnp.where(isj, _MASKV, rows)

    nchunks = (S + 127) // 128
    per_g = (nchunks + _G - 1) // _G
    stacks = [[jnp.full((_RB, 128), _NINF, jnp.float32)
               for _ in range(_D)] for _ in range(_G)]
    for g in range(_G):
        for q in range(per_g):
            c0 = (g * per_g + q) * 128
            if c0 >= S:
                break
            w = min(128, S - c0)
            c = x[:, c0:c0 + w]
            if w < 128:
                c = jnp.concatenate(
                    [c, jnp.full((_RB, 128 - w), _NINF, jnp.float32)], axis=1)
            st = stacks[g]
            for d in range(_D):
                hi = jnp.maximum(st[d], c)
                c = jnp.minimum(st[d], c)
                st[d] = hi

    ms = []
    for r in range(_K):
        top = stacks[0][0]
        for g in range(1, _G):
            top = jnp.maximum(top, stacks[g][0])
        m = jnp.max(top, axis=1, keepdims=True)
        ms.append(m)
        if r < _K - 1:
            for g in range(_G):
                st = stacks[g]
                hit = st[0] == m
                for d in range(_D - 1):
                    st[d] = jnp.where(hit, st[d + 1], st[d])
                st[_D - 1] = jnp.where(hit, _NINF, st[_D - 1])

    n = jnp.sum((x >= ms[_K - 1]).astype(jnp.float32), axis=1, keepdims=True)
    bad = jnp.sum(jnp.where(n == float(_K), 0.0, 1.0))

    h = jnp.zeros((_RB, 1), jnp.float32)
    for r in sorted(_SEL_RANKS):
        v = jnp.where(ms[r] == _MASKV, pos, ms[r])
        h += jnp.maximum(1.0 - pos + v, 0.0)
    part = jnp.sum(h)

    @pl.when(p == 0)
    def _init():
        acc[0] = 0.0
        acc[1] = 0.0

    acc[0] += part
    acc[1] += bad

    @pl.when(p == pl.num_programs(0) - 1)
    def _fin():
        _emit_outputs(acc, e_ref, m_ref, o_tot, o_c, o_f, P)
        o_bad[...] = jnp.reshape(acc[1], (1, 1))


def _exact_body(rowid_ref, *refs, S, P):
    sims = refs[:_RBX]
    j_ref, e_ref, m_ref = refs[_RBX:_RBX + 3]
    o_tot, o_c, o_f = refs[_RBX + 3:_RBX + 6]
    acc = refs[_RBX + 6]

    p = pl.program_id(0)
    rows = jnp.concatenate(
        [jnp.reshape(s[...], (1, S)) for s in sims], axis=0)
    jv = j_ref[...]
    iota = jax.lax.broadcasted_iota(jnp.int32, (_RBX, S), 1)
    isj = iota == jv
    pos = jnp.sum(jnp.where(isj, rows, 0.0), axis=1, keepdims=True)
    x = jnp.where(isj, _MASKV, rows)

    hinge = jnp.zeros((_RBX, 1), jnp.float32)
    for r in range(_K):
        m = jnp.max(x, axis=1, keepdims=True)
        if r in _SEL_RANKS:
            v = jnp.where(m == _MASKV, pos, m)
            hinge += jnp.maximum(1.0 - pos + v, 0.0)
        if r < _K - 1:
            idx = jnp.min(jnp.where(x == m, iota, S), axis=1, keepdims=True)
            x = jnp.where(iota == idx, -jnp.inf, x)
    part = jnp.sum(hinge)

    @pl.when(p == 0)
    def _init():
        acc[0] = 0.0

    acc[0] += part

    @pl.when(p == pl.num_programs(0) - 1)
    def _fin():
        _emit_outputs(acc, e_ref, m_ref, o_tot, o_c, o_f, P)


def _make_call(body, rb, n_out, S, P, n_acc):
    sim_spec = [
        pl.BlockSpec((1, 1, S), functools.partial(
            lambda gp, rid, r=0: (rid[rb * gp + r], 0, 0), r=r))
        for r in range(rb)
    ]
    in_specs = sim_spec + [
        pl.BlockSpec((rb, 1), lambda gp, rid: (gp, 0)),
        pl.BlockSpec((3, P), lambda gp, rid: (0, 0)),
        pl.BlockSpec((1, P), lambda gp, rid: (0, 0)),
    ]
    grid_spec = pltpu.PrefetchScalarGridSpec(
        num_scalar_prefetch=1,
        grid=(P // rb,),
        in_specs=in_specs,
        out_specs=[pl.BlockSpec((1, 1), lambda gp, rid: (0, 0))] * n_out,
        scratch_shapes=[pltpu.SMEM((n_acc,), jnp.float32)],
    )
    return pl.pallas_call(
        functools.partial(body, S=S, P=P),
        grid_spec=grid_spec,
        out_shape=[jax.ShapeDtypeStruct((1, 1), jnp.float32)] * n_out,
        compiler_params=pltpu.CompilerParams(
            dimension_semantics=("arbitrary",)),
    )


def kernel(sim_matrix, spv_b_ids, spv_i_ids, spv_j_ids, expec_f, gt_mask):
    B, L, S = sim_matrix.shape
    P = spv_b_ids.shape[0]
    sim3d = sim_matrix.reshape(B * L, 1, S)
    rowid = (spv_b_ids.astype(jnp.int32) * L + spv_i_ids.astype(jnp.int32))
    jcol = spv_j_ids.astype(jnp.int32).reshape(P, 1)
    expec_t = expec_f.astype(jnp.float32).T
    maskf = gt_mask.astype(jnp.float32).reshape(1, P)

    tot, lc, lf, bad = _make_call(_fast_body, _RB, 4, S, P, 2)(
        rowid, *([sim3d] * _RB), jcol, expec_t, maskf)

    def _use_fast(_):
        return tot[0, 0], lc[0, 0], lf[0, 0]

    def _run_exact(_):
        t, c, f = _make_call(_exact_body, _RBX, 3, S, P, 1)(
            rowid, *([sim3d] * _RBX), jcol, expec_t, maskf)
        return t[0, 0], c[0, 0], f[0, 0]

    tot_s, lc_s, lf_s = jax.lax.cond(bad[0, 0] == 0.0,
                                     _use_fast, _run_exact, 0)
    return (tot_s,
            jax.lax.stop_gradient(lc_s),
            jax.lax.stop_gradient(lf_s))


# speedup vs baseline: 1.1022x; 1.1022x over previous
import functools

import jax
import jax.numpy as jnp
from jax import lax
from jax.experimental import pallas as pl
from jax.experimental.pallas import tpu as pltpu
from jax.experimental.pallas import tpu_sc as plsc

_SEL_RANKS = frozenset((7, 4, 16, 19, 2, 5, 3, 6, 18, 10))

_K = 20
_NEG = 10
_MASKV = -1000000000.0
_NINF = float("-inf")
_RB = 32
_D = 4
_G = 4
_RBX = 8

_NC = 2
_NS = 16
_NW = _NC * _NS
_CH = 8


def _sc_gather_body(T, S, sim_hbm, idx_hbm, out_hbm,
                    idx_v, buf0, buf1, sem0, sem1):
    wid = lax.axis_index("s") * _NC + lax.axis_index("c")
    base = wid * (T * _CH)
    pltpu.sync_copy(idx_hbm.at[wid], idx_v)
    bufs = (buf0, buf1)
    sems = (sem0, sem1)
    cps = [None, None]
    cps[0] = pltpu.make_async_copy(
        sim_hbm.at[idx_v.at[0]], bufs[0], sems[0])
    cps[0].start()
    for t in range(T):
        cur = t % 2
        if t + 1 < T:
            cps[1 - cur] = pltpu.make_async_copy(
                sim_hbm.at[idx_v.at[t + 1]], bufs[1 - cur], sems[1 - cur])
            cps[1 - cur].start()
        cps[cur].wait()
        pltpu.sync_copy(bufs[cur], out_hbm.at[pl.ds(base + t * _CH, _CH)])


def _sc_gather(sim2d, rowid):
    BL, S = sim2d.shape
    P = rowid.shape[0]
    T = P // (_NW * _CH)
    idx3 = rowid.reshape(_NW, T, _CH)
    mesh = plsc.VectorSubcoreMesh(
        core_axis_name="c", subcore_axis_name="s",
        num_cores=_NC, num_subcores=_NS)
    f = pl.kernel(
        functools.partial(_sc_gather_body, T, S),
        out_type=jax.ShapeDtypeStruct((P, S), jnp.float32),
        mesh=mesh,
        compiler_params=pltpu.CompilerParams(use_tc_tiling_on_sc=False),
        scratch_types=[
            pltpu.VMEM((T, _CH), jnp.int32),
            pltpu.VMEM((_CH, S), jnp.float32),
            pltpu.VMEM((_CH, S), jnp.float32),
            pltpu.SemaphoreType.DMA,
            pltpu.SemaphoreType.DMA,
        ],
    )
    return f(sim2d, idx3)


def _fine_loss(e_ref, m_ref):
    e = e_ref[...]
    w = 1.0 / jnp.clip(e[2:3, :], 0.0001, None)
    per = w * (e[0:1, :] * e[0:1, :] + e[1:2, :] * e[1:2, :])
    mk = m_ref[...]
    return jnp.sum(per * mk) / jnp.maximum(jnp.sum(mk), 1.0)


def _emit_outputs(acc, e_ref, m_ref, o_tot, o_c, o_f, P):
    loss_c = acc[0] / (P * float(_NEG))
    loss_f = _fine_loss(e_ref, m_ref)
    o_tot[...] = jnp.reshape(1.0 * loss_c + 0.5 * loss_f, (1, 1))
    o_c[...] = jnp.reshape(loss_c, (1, 1))
    o_f[...] = jnp.reshape(loss_f, (1, 1))


def _fast_body(rows_ref, j_ref, e_ref, m_ref,
               o_tot, o_c, o_f, o_bad, acc, *, S, P):
    p = pl.program_id(0)

    rows = rows_ref[...]
    jv = j_ref[...]
    iota = jax.lax.broadcasted_iota(jnp.int32, (_RB, S), 1)
    isj = iota == jv
    pos = jnp.sum(jnp.where(isj, rows, 0.0), axis=1, keepdims=True)
    x = jnp.where(isj, _MASKV, rows)

    nchunks = (S + 127) // 128
    per_g = (nchunks + _G - 1) // _G
    stacks = [[jnp.full((_RB, 128), _NINF, jnp.float32)
               for _ in range(_D)] for _ in range(_G)]
    for g in range(_G):
        for q in range(per_g):
            c0 = (g * per_g + q) * 128
            if c0 >= S:
                break
            w = min(128, S - c0)
            c = x[:, c0:c0 + w]
            if w < 128:
                c = jnp.concatenate(
                    [c, jnp.full((_RB, 128 - w), _NINF, jnp.float32)], axis=1)
            st = stacks[g]
            for d in range(_D):
                hi = jnp.maximum(st[d], c)
                c = jnp.minimum(st[d], c)
                st[d] = hi

    ms = []
    for r in range(_K):
        top = stacks[0][0]
        for g in range(1, _G):
            top = jnp.maximum(top, stacks[g][0])
        m = jnp.max(top, axis=1, keepdims=True)
        ms.append(m)
        if r < _K - 1:
            for g in range(_G):
                st = stacks[g]
                hit = st[0] == m
                for d in range(_D - 1):
                    st[d] = jnp.where(hit, st[d + 1], st[d])
                st[_D - 1] = jnp.where(hit, _NINF, st[_D - 1])

    n = jnp.sum((x >= ms[_K - 1]).astype(jnp.float32), axis=1, keepdims=True)
    bad = jnp.sum(jnp.where(n == float(_K), 0.0, 1.0))

    h = jnp.zeros((_RB, 1), jnp.float32)
    for r in sorted(_SEL_RANKS):
        v = jnp.where(ms[r] == _MASKV, pos, ms[r])
        h += jnp.maximum(1.0 - pos + v, 0.0)
    part = jnp.sum(h)

    @pl.when(p == 0)
    def _init():
        acc[0] = 0.0
        acc[1] = 0.0

    acc[0] += part
    acc[1] += bad

    @pl.when(p == pl.num_programs(0) - 1)
    def _fin():
        _emit_outputs(acc, e_ref, m_ref, o_tot, o_c, o_f, P)
        o_bad[...] = jnp.reshape(acc[1], (1, 1))


def _tc_fast(staged, jcol, expec_t, maskf, S, P):
    in_specs = [
        pl.BlockSpec((_RB, S), lambda gp: (gp, 0)),
        pl.BlockSpec((_RB, 1), lambda gp: (gp, 0)),
        pl.BlockSpec((3, P), lambda gp: (0, 0)),
        pl.BlockSpec((1, P), lambda gp: (0, 0)),
    ]
    return pl.pallas_call(
        functools.partial(_fast_body, S=S, P=P),
        grid=(P // _RB,),
        in_specs=in_specs,
        out_specs=[pl.BlockSpec((1, 1), lambda gp: (0, 0))] * 4,
        out_shape=[jax.ShapeDtypeStruct((1, 1), jnp.float32)] * 4,
        scratch_shapes=[pltpu.SMEM((2,), jnp.float32)],
        compiler_params=pltpu.CompilerParams(
            dimension_semantics=("arbitrary",)),
    )(staged, jcol, expec_t, maskf)


def _exact_body(rowid_ref, *refs, S, P):
    sims = refs[:_RBX]
    j_ref, e_ref, m_ref = refs[_RBX:_RBX + 3]
    o_tot, o_c, o_f = refs[_RBX + 3:_RBX + 6]
    acc = refs[_RBX + 6]

    p = pl.program_id(0)
    rows = jnp.concatenate(
        [jnp.reshape(s[...], (1, S)) for s in sims], axis=0)
    jv = j_ref[...]
    iota = jax.lax.broadcasted_iota(jnp.int32, (_RBX, S), 1)
    isj = iota == jv
    pos = jnp.sum(jnp.where(isj, rows, 0.0), axis=1, keepdims=True)
    x = jnp.where(isj, _MASKV, rows)

    hinge = jnp.zeros((_RBX, 1), jnp.float32)
    for r in range(_K):
        m = jnp.max(x, axis=1, keepdims=True)
        if r in _SEL_RANKS:
            v = jnp.where(m == _MASKV, pos, m)
            hinge += jnp.maximum(1.0 - pos + v, 0.0)
        if r < _K - 1:
            idx = jnp.min(jnp.where(x == m, iota, S), axis=1, keepdims=True)
            x = jnp.where(iota == idx, -jnp.inf, x)
    part = jnp.sum(hinge)

    @pl.when(p == 0)
    def _init():
        acc[0] = 0.0

    acc[0] += part

    @pl.when(p == pl.num_programs(0) - 1)
    def _fin():
        _emit_outputs(acc, e_ref, m_ref, o_tot, o_c, o_f, P)


def _tc_exact(sim3d, rowid, jcol, expec_t, maskf, S, P):
    sim_spec = [
        pl.BlockSpec((1, 1, S), functools.partial(
            lambda gp, rid, r=0: (rid[_RBX * gp + r], 0, 0), r=r))
        for r in range(_RBX)
    ]
    in_specs = sim_spec + [
        pl.BlockSpec((_RBX, 1), lambda gp, rid: (gp, 0)),
        pl.BlockSpec((3, P), lambda gp, rid: (0, 0)),
        pl.BlockSpec((1, P), lambda gp, rid: (0, 0)),
    ]
    grid_spec = pltpu.PrefetchScalarGridSpec(
        num_scalar_prefetch=1,
        grid=(P // _RBX,),
        in_specs=in_specs,
        out_specs=[pl.BlockSpec((1, 1), lambda gp, rid: (0, 0))] * 3,
        scratch_shapes=[pltpu.SMEM((1,), jnp.float32)],
    )
    return pl.pallas_call(
        functools.partial(_exact_body, S=S, P=P),
        grid_spec=grid_spec,
        out_shape=[jax.ShapeDtypeStruct((1, 1), jnp.float32)] * 3,
        compiler_params=pltpu.CompilerParams(
            dimension_semantics=("arbitrary",)),
    )(rowid, *([sim3d] * _RBX), jcol, expec_t, maskf)


def kernel(sim_matrix, spv_b_ids, spv_i_ids, spv_j_ids, expec_f, gt_mask):
    B, L, S = sim_matrix.shape
    P = spv_b_ids.shape[0]
    sim2d = sim_matrix.reshape(B * L, S)
    rowid = (spv_b_ids.astype(jnp.int32) * L + spv_i_ids.astype(jnp.int32))
    jcol = spv_j_ids.astype(jnp.int32).reshape(P, 1)
    expec_t = expec_f.astype(jnp.float32).T
    maskf = gt_mask.astype(jnp.float32).reshape(1, P)

    staged = _sc_gather(sim2d, rowid)
    tot, lc, lf, bad = _tc_fast(staged, jcol, expec_t, maskf, S, P)

    def _use_fast(_):
        return tot[0, 0], lc[0, 0], lf[0, 0]

    def _run_exact(_):
        t, c, f = _tc_exact(sim2d.reshape(B * L, 1, S), rowid, jcol,
                            expec_t, maskf, S, P)
        return t[0, 0], c[0, 0], f[0, 0]

    tot_s, lc_s, lf_s = jax.lax.cond(bad[0, 0] == 0.0,
                                     _use_fast, _run_exact, 0)
    return (tot_s,
            jax.lax.stop_gradient(lc_s),
            jax.lax.stop_gradient(lf_s))
